# baseline (speedup 1.0000x reference)
import jax
import jax.numpy as jnp
from jax import lax
from jax.experimental import pallas as pl
from jax.experimental.pallas import tpu as pltpu
from jax.experimental.pallas import tpu_sc as plsc

N = 10000
E = 320000
D = 128
HID = 256
EPS = 1e-5

NC = 2
NS = 16
CH = 64
SB = 32
NBLK = 10
NBUF = 4
EPT = NBLK * SB * CH
EPC = NS * EPT
NP = 10240
RPT = NP // NS
ZR = 32

_HIGH = lax.Precision.HIGHEST


def _dot(a, b):
    return jnp.dot(a, b, preferred_element_type=jnp.float32, precision=_HIGH)


def _mesh():
    return plsc.VectorSubcoreMesh(core_axis_name="c", subcore_axis_name="s",
                                  num_cores=NC, num_subcores=NS)


def _aggregate(ytab, gidx, sidx):
    def body(ytab_ref, gidx_ref, sidx_ref, agg_out, gslab, sslab, *refs):
        bufs = refs[:NBUF]
        zrow = refs[NBUF]
        agg_sh = refs[NBUF + 1]
        gsems = refs[NBUF + 2:2 * NBUF + 2]
        ssems = refs[2 * NBUF + 2:]
        c = lax.axis_index("c")
        s = lax.axis_index("s")

        z16 = jnp.zeros((16,), jnp.float32)
        for i in range(ZR):
            for k in range(D // 16):
                zrow[i, pl.ds(k * 16, 16)] = z16

        base = s * RPT
        for t in range(RPT // ZR):
            pltpu.sync_copy(zrow, agg_sh.at[pl.ds(base + t * ZR, ZR)])
        plsc.subcore_barrier()

        def blk_step(blk, carry):
            pltpu.sync_copy(gidx_ref.at[c, s, blk], gslab)
            pltpu.sync_copy(sidx_ref.at[c, s, blk], sslab)
            gd = [None] * SB
            sd = [None] * SB
            for p in range(3):
                gd[p] = pltpu.async_copy(ytab_ref.at[gslab.at[p]], bufs[p],
                                         gsems[p])
            for j in range(SB):
                gd[j].wait()
                sd[j] = pltpu.async_copy(bufs[j % NBUF],
                                         agg_sh.at[sslab.at[j]],
                                         ssems[j % NBUF], add=True)
                if j >= 1:
                    sd[j - 1].wait()
                if j + 3 < SB:
                    b = (j + 3) % NBUF
                    gd[j + 3] = pltpu.async_copy(
                        ytab_ref.at[gslab.at[j + 3]], bufs[b], gsems[b])
            sd[SB - 1].wait()
            return carry

        lax.fori_loop(0, NBLK, blk_step, 0)
        plsc.subcore_barrier()
        pltpu.sync_copy(agg_sh.at[pl.ds(base, RPT)],
                        agg_out.at[c, pl.ds(base, RPT)])

    return pl.kernel(
        body,
        out_type=jax.ShapeDtypeStruct((NC, NP, D), jnp.float32),
        mesh=_mesh(),
        scratch_types=[
            pltpu.VMEM((SB, CH), jnp.int32),
            pltpu.VMEM((SB, CH), jnp.int32),
        ] + [pltpu.VMEM((CH, D), jnp.float32) for _ in range(NBUF)]
        + [
            pltpu.VMEM((ZR, D), jnp.float32),
            pltpu.VMEM_SHARED((NP, D), jnp.float32),
        ] + [pltpu.SemaphoreType.DMA for _ in range(2 * NBUF)],
    )(ytab, gidx, sidx)


SBC = 16


def _counts(sidxc):
    def body(sidx_ref, cnt_out, sslab, ones, zrow, cnt_sh, sem):
        c = lax.axis_index("c")
        s = lax.axis_index("s")

        z16 = jnp.zeros((16,), jnp.float32)
        o16 = jnp.ones((16,), jnp.float32)
        for i in range(ZR):
            for k in range(D // 16):
                zrow[i, pl.ds(k * 16, 16)] = z16
        for i in range(2 * CH):
            for k in range(D // 16):
                ones[i, pl.ds(k * 16, 16)] = o16

        base = s * RPT
        for t in range(RPT // ZR):
            pltpu.sync_copy(zrow, cnt_sh.at[pl.ds(base + t * ZR, ZR)])
        plsc.subcore_barrier()

        def blk_step(blk, carry):
            pltpu.sync_copy(sidx_ref.at[c, s, blk], sslab)
            sd = [pltpu.async_copy(ones, cnt_sh.at[sslab.at[j]], sem,
                                   add=True)
                  for j in range(SBC)]
            for d in sd:
                d.wait()
            return carry

        lax.fori_loop(0, NBLK, blk_step, 0)
        plsc.subcore_barrier()
        pltpu.sync_copy(cnt_sh.at[pl.ds(base, RPT)],
                        cnt_out.at[c, pl.ds(base, RPT)])

    return pl.kernel(
        body,
        out_type=jax.ShapeDtypeStruct((NC, NP, D), jnp.float32),
        mesh=_mesh(),
        scratch_types=[
            pltpu.VMEM((SBC, 2 * CH), jnp.int32),
            pltpu.VMEM((2 * CH, D), jnp.float32),
            pltpu.VMEM((ZR, D), jnp.float32),
            pltpu.VMEM_SHARED((NP, D), jnp.float32),
            pltpu.SemaphoreType.DMA,
        ],
    )(sidxc)


_NB = 10
_BLK = N // _NB


def _tc_pre(x, wfl, wbl, wfr, wbr):
    def body(x_ref, wfl_ref, wbl_ref, wfr_ref, wbr_ref, y_ref, r_ref):
        xb = x_ref[...]
        y_ref[0] = _dot(xb, wfl_ref[...])
        y_ref[1] = _dot(xb, wbl_ref[...])
        r_ref[0] = _dot(xb, wfr_ref[...])
        r_ref[1] = _dot(xb, wbr_ref[...])

    w_spec = pl.BlockSpec((D, D), lambda i: (0, 0))
    return pl.pallas_call(
        body,
        grid=(_NB,),
        in_specs=[pl.BlockSpec((_BLK, D), lambda i: (i, 0)),
                  w_spec, w_spec, w_spec, w_spec],
        out_specs=[pl.BlockSpec((2, _BLK, D), lambda i: (0, i, 0)),
                   pl.BlockSpec((2, _BLK, D), lambda i: (0, i, 0))],
        out_shape=[jax.ShapeDtypeStruct((2, N, D), jnp.float32),
                   jax.ShapeDtypeStruct((2, N, D), jnp.float32)],
    )(x, wfl, wbl, wfr, wbr)


def _bn_relu(h, s, q, g, b):
    m = s / N
    v = q / N - m * m
    return jnp.maximum((h - m) * lax.rsqrt(v + EPS) * g + b, 0.0)


def _stats_phase(i, agg_ref, cnt_ref, rtab_ref, bf_ref, bb_ref,
                 hbuf, sbuf, qbuf):
    cin = jnp.maximum(cnt_ref[0, :, 0:1], 1.0)
    cout = jnp.maximum(cnt_ref[1, :, 0:1], 1.0)
    hf = agg_ref[0] / cin + bf_ref[...] + rtab_ref[0]
    hb = agg_ref[1] / cout + bb_ref[...] + rtab_ref[1]
    h = jnp.concatenate([hf, hb], axis=1)
    hbuf[pl.ds(i * _BLK, _BLK), :] = h

    @pl.when(i == 0)
    def _():
        sbuf[...] = jnp.zeros_like(sbuf)
        qbuf[...] = jnp.zeros_like(qbuf)

    sbuf[...] += jnp.sum(h, axis=0, keepdims=True)
    qbuf[...] += jnp.sum(h * h, axis=0, keepdims=True)


def _tc_mid(agg, cnt, rtab, bf, bb, g, b, wfl, wbl, wfr, wbr):
    def body(agg_ref, cnt_ref, rtab_ref, bf_ref, bb_ref, g_ref, b_ref,
             wfl_ref, wbl_ref, wfr_ref, wbr_ref, y2_ref, r2_ref,
             hbuf, sbuf, qbuf):
        i = pl.program_id(0)

        @pl.when(i < _NB)
        def _():
            _stats_phase(i, agg_ref, cnt_ref, rtab_ref, bf_ref, bb_ref,
                         hbuf, sbuf, qbuf)

        @pl.when(i >= _NB)
        def _():
            h = hbuf[pl.ds((i - _NB) * _BLK, _BLK), :]
            hn = _bn_relu(h, sbuf[...], qbuf[...], g_ref[...], b_ref[...])
            y2_ref[0] = _dot(hn, wfl_ref[...])
            y2_ref[1] = _dot(hn, wbl_ref[...])
            r2_ref[0] = _dot(hn, wfr_ref[...])
            r2_ref[1] = _dot(hn, wbr_ref[...])

    b_spec = pl.BlockSpec((1, D), lambda i: (0, 0))
    stat_spec = pl.BlockSpec((1, HID), lambda i: (0, 0))
    w_spec = pl.BlockSpec((HID, D), lambda i: (0, 0))
    blk3 = pl.BlockSpec((2, _BLK, D), lambda i: (0, i % _NB, 0))
    return pl.pallas_call(
        body,
        grid=(2 * _NB,),
        in_specs=[blk3, blk3, blk3, b_spec, b_spec, stat_spec, stat_spec,
                  w_spec, w_spec, w_spec, w_spec],
        out_specs=[blk3, blk3],
        out_shape=[jax.ShapeDtypeStruct((2, N, D), jnp.float32),
                   jax.ShapeDtypeStruct((2, N, D), jnp.float32)],
        scratch_shapes=[pltpu.VMEM((N, HID), jnp.float32),
                        pltpu.VMEM((1, HID), jnp.float32),
                        pltpu.VMEM((1, HID), jnp.float32)],
    )(agg, cnt, rtab, bf, bb, g, b, wfl, wbl, wfr, wbr)


def _tc_post(agg, cnt, rtab, bf, bb, g, b, wp, bp):
    def body(agg_ref, cnt_ref, rtab_ref, bf_ref, bb_ref, g_ref, b_ref,
             wp_ref, bp_ref, embs_ref, genc_ref, hbuf, sbuf, qbuf):
        i = pl.program_id(0)

        @pl.when(i < _NB)
        def _():
            _stats_phase(i, agg_ref, cnt_ref, rtab_ref, bf_ref, bb_ref,
                         hbuf, sbuf, qbuf)

        @pl.when(i >= _NB)
        def _():
            h = hbuf[pl.ds((i - _NB) * _BLK, _BLK), :]
            embs = _bn_relu(h, sbuf[...], qbuf[...], g_ref[...], b_ref[...])
            embs_ref[...] = embs
            proj = jnp.maximum(_dot(embs, wp_ref[...]) + bp_ref[...], 0.0)
            pmax = jnp.max(proj, axis=0, keepdims=True)

            @pl.when(i == _NB)
            def _():
                genc_ref[...] = jnp.zeros_like(genc_ref)

            genc_ref[...] = jnp.maximum(genc_ref[...], pmax)

    b_spec = pl.BlockSpec((1, D), lambda i: (0, 0))
    stat_spec = pl.BlockSpec((1, HID), lambda i: (0, 0))
    blk3 = pl.BlockSpec((2, _BLK, D), lambda i: (0, i % _NB, 0))
    return pl.pallas_call(
        body,
        grid=(2 * _NB,),
        in_specs=[blk3, blk3, blk3, b_spec, b_spec, stat_spec, stat_spec,
                  pl.BlockSpec((HID, HID), lambda i: (0, 0)), stat_spec],
        out_specs=[pl.BlockSpec((_BLK, HID), lambda i: (i % _NB, 0)),
                   stat_spec],
        out_shape=[jax.ShapeDtypeStruct((N, HID), jnp.float32),
                   jax.ShapeDtypeStruct((1, HID), jnp.float32)],
        scratch_shapes=[pltpu.VMEM((N, HID), jnp.float32),
                        pltpu.VMEM((1, HID), jnp.float32),
                        pltpu.VMEM((1, HID), jnp.float32)],
    )(agg, cnt, rtab, bf, bb, g, b, wp, bp)


def kernel(x, edge_index, W1f_l, b1f, W1f_r, W1b_l, b1b, W1b_r, bn1_g, bn1_b,
           W2f_l, b2f, W2f_r, W2b_l, b2b, W2b_r, bn2_g, bn2_b, Wp, bp):
    src = edge_index[0]
    dst = edge_index[1]
    pad = EPC - E
    zpad = jnp.zeros((pad,), jnp.int32)
    tpad = jnp.full((pad,), N, jnp.int32)
    gidx = jnp.stack([jnp.concatenate([src, zpad]),
                      jnp.concatenate([dst + N, zpad])])
    sidx = jnp.stack([jnp.concatenate([dst, tpad]),
                      jnp.concatenate([src, tpad])])
    gidx = gidx.reshape(NC, NS, NBLK, SB, CH)
    sidx = sidx.reshape(NC, NS, NBLK, SB, CH)

    cnt = _counts(sidx.reshape(NC, NS, NBLK, SBC, 2 * CH))
    ytab1, rtab1 = _tc_pre(x, W1f_l, W1b_l, W1f_r, W1b_r)
    agg1 = _aggregate(ytab1.reshape(NC * N, D), gidx, sidx)
    ytab2, rtab2 = _tc_mid(agg1, cnt, rtab1,
                           b1f.reshape(1, D), b1b.reshape(1, D),
                           bn1_g.reshape(1, HID), bn1_b.reshape(1, HID),
                           W2f_l, W2b_l, W2f_r, W2b_r)
    agg2 = _aggregate(ytab2.reshape(NC * N, D), gidx, sidx)
    node_embs, genc = _tc_post(agg2, cnt, rtab2,
                               b2f.reshape(1, D), b2b.reshape(1, D),
                               bn2_g.reshape(1, HID), bn2_b.reshape(1, HID),
                               Wp, bp.reshape(1, HID))
    return (node_embs, genc.reshape(HID))


# speedup vs baseline: 1.0029x; 1.0029x over previous
import jax
import jax.numpy as jnp
from jax import lax
from jax.experimental import pallas as pl
from jax.experimental.pallas import tpu as pltpu
from jax.experimental.pallas import tpu_sc as plsc

N = 10000
E = 320000
D = 128
HID = 256
EPS = 1e-5

NC = 2
NS = 16
CH = 64
SB = 32
NBLK = 10
NBUF = 4
EPT = NBLK * SB * CH
EPC = NS * EPT
NP = 10240
RPT = NP // NS
ZR = 32

_HIGH = lax.Precision.HIGHEST


def _dot(a, b):
    return jnp.dot(a, b, preferred_element_type=jnp.float32, precision=_HIGH)


def _mesh():
    return plsc.VectorSubcoreMesh(core_axis_name="c", subcore_axis_name="s",
                                  num_cores=NC, num_subcores=NS)


def _aggregate(ytab, gidx, sidx):
    def body(ytab_ref, gidx_ref, sidx_ref, agg_out, gslab, sslab, *refs):
        bufs = refs[:NBUF]
        zrow = refs[NBUF]
        agg_sh = refs[NBUF + 1]
        gsems = refs[NBUF + 2:2 * NBUF + 2]
        ssems = refs[2 * NBUF + 2:]
        c = lax.axis_index("c")
        s = lax.axis_index("s")

        z16 = jnp.zeros((16,), jnp.float32)
        for i in range(ZR):
            for k in range(D // 16):
                zrow[i, pl.ds(k * 16, 16)] = z16

        base = s * RPT
        for t in range(RPT // ZR):
            pltpu.sync_copy(zrow, agg_sh.at[pl.ds(base + t * ZR, ZR)])
        plsc.subcore_barrier()

        def blk_step(blk, carry):
            pltpu.sync_copy(gidx_ref.at[c, s, blk], gslab)
            pltpu.sync_copy(sidx_ref.at[c, s, blk], sslab)
            gd = [None] * SB
            sd = [None] * SB
            for p in range(3):
                gd[p] = pltpu.async_copy(ytab_ref.at[gslab.at[p]], bufs[p],
                                         gsems[p])
            for j in range(SB):
                gd[j].wait()
                sd[j] = pltpu.async_copy(bufs[j % NBUF],
                                         agg_sh.at[sslab.at[j]],
                                         ssems[j % NBUF], add=True)
                if j >= 1:
                    sd[j - 1].wait()
                if j + 3 < SB:
                    b = (j + 3) % NBUF
                    gd[j + 3] = pltpu.async_copy(
                        ytab_ref.at[gslab.at[j + 3]], bufs[b], gsems[b])
            sd[SB - 1].wait()
            return carry

        lax.fori_loop(0, NBLK, blk_step, 0)
        plsc.subcore_barrier()
        pltpu.sync_copy(agg_sh.at[pl.ds(base, RPT)],
                        agg_out.at[c, pl.ds(base, RPT)])

    return pl.kernel(
        body,
        out_type=jax.ShapeDtypeStruct((NC, NP, D), jnp.float32),
        mesh=_mesh(),
        scratch_types=[
            pltpu.VMEM((SB, CH), jnp.int32),
            pltpu.VMEM((SB, CH), jnp.int32),
        ] + [pltpu.VMEM((CH, D), jnp.float32) for _ in range(NBUF)]
        + [
            pltpu.VMEM((ZR, D), jnp.float32),
            pltpu.VMEM_SHARED((NP, D), jnp.float32),
        ] + [pltpu.SemaphoreType.DMA for _ in range(2 * NBUF)],
    )(ytab, gidx, sidx)


SBC = 16


def _counts(sidxc):
    def body(sidx_ref, cnt_out, sslab, ones, zrow, cnt_sh, sem):
        c = lax.axis_index("c")
        s = lax.axis_index("s")

        z16 = jnp.zeros((16,), jnp.float32)
        o16 = jnp.ones((16,), jnp.float32)
        for i in range(ZR):
            for k in range(D // 16):
                zrow[i, pl.ds(k * 16, 16)] = z16
        for i in range(2 * CH):
            for k in range(D // 16):
                ones[i, pl.ds(k * 16, 16)] = o16

        base = s * RPT
        for t in range(RPT // ZR):
            pltpu.sync_copy(zrow, cnt_sh.at[pl.ds(base + t * ZR, ZR)])
        plsc.subcore_barrier()

        def blk_step(blk, carry):
            pltpu.sync_copy(sidx_ref.at[c, s, blk], sslab)
            sd = [pltpu.async_copy(ones, cnt_sh.at[sslab.at[j]], sem,
                                   add=True)
                  for j in range(SBC)]
            for d in sd:
                d.wait()
            return carry

        lax.fori_loop(0, NBLK, blk_step, 0)
        plsc.subcore_barrier()
        pltpu.sync_copy(cnt_sh.at[pl.ds(base, RPT)],
                        cnt_out.at[c, pl.ds(base, RPT)])

    return pl.kernel(
        body,
        out_type=jax.ShapeDtypeStruct((NC, NP, D), jnp.float32),
        mesh=_mesh(),
        scratch_types=[
            pltpu.VMEM((SBC, 2 * CH), jnp.int32),
            pltpu.VMEM((2 * CH, D), jnp.float32),
            pltpu.VMEM((ZR, D), jnp.float32),
            pltpu.VMEM_SHARED((NP, D), jnp.float32),
            pltpu.SemaphoreType.DMA,
        ],
    )(sidxc)


_NB = 10
_BLK = N // _NB


def _tc_pre(x, wfl, wbl, wfr, wbr):
    def body(x_ref, wfl_ref, wbl_ref, wfr_ref, wbr_ref, y_ref, r_ref):
        xb = x_ref[...]
        y_ref[0] = _dot(xb, wfl_ref[...])
        y_ref[1] = _dot(xb, wbl_ref[...])
        r_ref[0] = _dot(xb, wfr_ref[...])
        r_ref[1] = _dot(xb, wbr_ref[...])

    w_spec = pl.BlockSpec((D, D), lambda i: (0, 0))
    return pl.pallas_call(
        body,
        grid=(_NB,),
        in_specs=[pl.BlockSpec((_BLK, D), lambda i: (i, 0)),
                  w_spec, w_spec, w_spec, w_spec],
        out_specs=[pl.BlockSpec((2, _BLK, D), lambda i: (0, i, 0)),
                   pl.BlockSpec((2, _BLK, D), lambda i: (0, i, 0))],
        out_shape=[jax.ShapeDtypeStruct((2, N, D), jnp.float32),
                   jax.ShapeDtypeStruct((2, N, D), jnp.float32)],
    )(x, wfl, wbl, wfr, wbr)


def _bn_relu(h, s, q, g, b):
    m = s / N
    v = q / N - m * m
    return jnp.maximum((h - m) * lax.rsqrt(v + EPS) * g + b, 0.0)


def _stats_phase(i, agg_ref, cnt_ref, rtab_ref, bf_ref, bb_ref,
                 hbuf, sbuf, qbuf):
    cin = jnp.maximum(cnt_ref[0, :, 0:1], 1.0)
    cout = jnp.maximum(cnt_ref[1, :, 0:1], 1.0)
    hf = agg_ref[0] / cin + bf_ref[...] + rtab_ref[0]
    hb = agg_ref[1] / cout + bb_ref[...] + rtab_ref[1]
    h = jnp.concatenate([hf, hb], axis=1)
    hbuf[pl.ds(i * _BLK, _BLK), :] = h

    @pl.when(i == 0)
    def _():
        sbuf[...] = jnp.zeros_like(sbuf)
        qbuf[...] = jnp.zeros_like(qbuf)

    sbuf[...] += jnp.sum(h, axis=0, keepdims=True)
    qbuf[...] += jnp.sum(h * h, axis=0, keepdims=True)


def _tc_mid(agg, cnt, rtab, bf, bb, g, b, wfl, wbl, wfr, wbr):
    def body(agg_ref, cnt_ref, rtab_ref, bf_ref, bb_ref, g_ref, b_ref,
             wfl_ref, wbl_ref, wfr_ref, wbr_ref, y2_ref, r2_ref,
             hbuf, sbuf, qbuf):
        i = pl.program_id(0)

        @pl.when(i < _NB)
        def _():
            _stats_phase(i, agg_ref, cnt_ref, rtab_ref, bf_ref, bb_ref,
                         hbuf, sbuf, qbuf)

        @pl.when(i >= _NB)
        def _():
            h = hbuf[pl.ds((i - _NB) * _BLK, _BLK), :]
            hn = _bn_relu(h, sbuf[...], qbuf[...], g_ref[...], b_ref[...])
            y2_ref[0] = _dot(hn, wfl_ref[...])
            y2_ref[1] = _dot(hn, wbl_ref[...])
            r2_ref[0] = _dot(hn, wfr_ref[...])
            r2_ref[1] = _dot(hn, wbr_ref[...])

    b_spec = pl.BlockSpec((1, D), lambda i: (0, 0))
    stat_spec = pl.BlockSpec((1, HID), lambda i: (0, 0))
    w_spec = pl.BlockSpec((HID, D), lambda i: (0, 0))
    blk3i = pl.BlockSpec((2, _BLK, D), lambda i: (0, jnp.where(i < _NB, i, 0), 0))
    blk3o = pl.BlockSpec((2, _BLK, D), lambda i: (0, i % _NB, 0))
    return pl.pallas_call(
        body,
        grid=(2 * _NB,),
        in_specs=[blk3i, blk3i, blk3i, b_spec, b_spec, stat_spec, stat_spec,
                  w_spec, w_spec, w_spec, w_spec],
        out_specs=[blk3o, blk3o],
        out_shape=[jax.ShapeDtypeStruct((2, N, D), jnp.float32),
                   jax.ShapeDtypeStruct((2, N, D), jnp.float32)],
        scratch_shapes=[pltpu.VMEM((N, HID), jnp.float32),
                        pltpu.VMEM((1, HID), jnp.float32),
                        pltpu.VMEM((1, HID), jnp.float32)],
    )(agg, cnt, rtab, bf, bb, g, b, wfl, wbl, wfr, wbr)


def _tc_post(agg, cnt, rtab, bf, bb, g, b, wp, bp):
    def body(agg_ref, cnt_ref, rtab_ref, bf_ref, bb_ref, g_ref, b_ref,
             wp_ref, bp_ref, embs_ref, genc_ref, hbuf, sbuf, qbuf):
        i = pl.program_id(0)

        @pl.when(i < _NB)
        def _():
            _stats_phase(i, agg_ref, cnt_ref, rtab_ref, bf_ref, bb_ref,
                         hbuf, sbuf, qbuf)

        @pl.when(i >= _NB)
        def _():
            h = hbuf[pl.ds((i - _NB) * _BLK, _BLK), :]
            embs = _bn_relu(h, sbuf[...], qbuf[...], g_ref[...], b_ref[...])
            embs_ref[...] = embs
            proj = jnp.maximum(_dot(embs, wp_ref[...]) + bp_ref[...], 0.0)
            pmax = jnp.max(proj, axis=0, keepdims=True)

            @pl.when(i == _NB)
            def _():
                genc_ref[...] = jnp.zeros_like(genc_ref)

            genc_ref[...] = jnp.maximum(genc_ref[...], pmax)

    b_spec = pl.BlockSpec((1, D), lambda i: (0, 0))
    stat_spec = pl.BlockSpec((1, HID), lambda i: (0, 0))
    blk3i = pl.BlockSpec((2, _BLK, D), lambda i: (0, jnp.where(i < _NB, i, 0), 0))
    return pl.pallas_call(
        body,
        grid=(2 * _NB,),
        in_specs=[blk3i, blk3i, blk3i, b_spec, b_spec, stat_spec, stat_spec,
                  pl.BlockSpec((HID, HID), lambda i: (0, 0)), stat_spec],
        out_specs=[pl.BlockSpec((_BLK, HID), lambda i: (i % _NB, 0)),
                   stat_spec],
        out_shape=[jax.ShapeDtypeStruct((N, HID), jnp.float32),
                   jax.ShapeDtypeStruct((1, HID), jnp.float32)],
        scratch_shapes=[pltpu.VMEM((N, HID), jnp.float32),
                        pltpu.VMEM((1, HID), jnp.float32),
                        pltpu.VMEM((1, HID), jnp.float32)],
    )(agg, cnt, rtab, bf, bb, g, b, wp, bp)


def kernel(x, edge_index, W1f_l, b1f, W1f_r, W1b_l, b1b, W1b_r, bn1_g, bn1_b,
           W2f_l, b2f, W2f_r, W2b_l, b2b, W2b_r, bn2_g, bn2_b, Wp, bp):
    src = edge_index[0]
    dst = edge_index[1]
    pad = EPC - E
    zpad = jnp.zeros((pad,), jnp.int32)
    tpad = jnp.full((pad,), N, jnp.int32)
    gidx = jnp.stack([jnp.concatenate([src, zpad]),
                      jnp.concatenate([dst + N, zpad])])
    sidx = jnp.stack([jnp.concatenate([dst, tpad]),
                      jnp.concatenate([src, tpad])])
    gidx = gidx.reshape(NC, NS, NBLK, SB, CH)
    sidx = sidx.reshape(NC, NS, NBLK, SB, CH)

    cnt = _counts(sidx.reshape(NC, NS, NBLK, SBC, 2 * CH))
    ytab1, rtab1 = _tc_pre(x, W1f_l, W1b_l, W1f_r, W1b_r)
    agg1 = _aggregate(ytab1.reshape(NC * N, D), gidx, sidx)
    ytab2, rtab2 = _tc_mid(agg1, cnt, rtab1,
                           b1f.reshape(1, D), b1b.reshape(1, D),
                           bn1_g.reshape(1, HID), bn1_b.reshape(1, HID),
                           W2f_l, W2b_l, W2f_r, W2b_r)
    agg2 = _aggregate(ytab2.reshape(NC * N, D), gidx, sidx)
    node_embs, genc = _tc_post(agg2, cnt, rtab2,
                               b2f.reshape(1, D), b2b.reshape(1, D),
                               bn2_g.reshape(1, HID), bn2_b.reshape(1, HID),
                               Wp, bp.reshape(1, HID))
    return (node_embs, genc.reshape(HID))


# speedup vs baseline: 1.0039x; 1.0010x over previous
import jax
import jax.numpy as jnp
from jax import lax
from jax.experimental import pallas as pl
from jax.experimental.pallas import tpu as pltpu
from jax.experimental.pallas import tpu_sc as plsc

N = 10000
E = 320000
D = 128
HID = 256
EPS = 1e-5

NC = 2
NS = 16
CH = 64
SB = 32
NBLK = 10
NBUF = 4
EPT = NBLK * SB * CH
EPC = NS * EPT
NP = 10240
RPT = NP // NS
ZR = 32

_HIGH = lax.Precision.HIGHEST


def _dot(a, b):
    return jnp.dot(a, b, preferred_element_type=jnp.float32, precision=_HIGH)


def _mesh():
    return plsc.VectorSubcoreMesh(core_axis_name="c", subcore_axis_name="s",
                                  num_cores=NC, num_subcores=NS)


def _aggregate(ytab, gidx, sidx):
    def body(ytab_ref, gidx_ref, sidx_ref, agg_out, gslab, sslab, *refs):
        bufs = refs[:NBUF]
        zrow = refs[NBUF]
        agg_sh = refs[NBUF + 1]
        gsems = refs[NBUF + 2:2 * NBUF + 2]
        ssems = refs[2 * NBUF + 2:]
        c = lax.axis_index("c")
        s = lax.axis_index("s")

        z16 = jnp.zeros((16,), jnp.float32)
        for i in range(ZR):
            for k in range(D // 16):
                zrow[i, pl.ds(k * 16, 16)] = z16

        base = s * RPT
        for t in range(RPT // ZR):
            pltpu.sync_copy(zrow, agg_sh.at[pl.ds(base + t * ZR, ZR)])
        plsc.subcore_barrier()

        def blk_step(blk, carry):
            pltpu.sync_copy(gidx_ref.at[c, s, blk], gslab)
            pltpu.sync_copy(sidx_ref.at[c, s, blk], sslab)
            gd = [None] * SB
            sd = [None] * SB
            for p in range(3):
                gd[p] = pltpu.async_copy(ytab_ref.at[gslab.at[p]], bufs[p],
                                         gsems[p])
            for j in range(SB):
                gd[j].wait()
                sd[j] = pltpu.async_copy(bufs[j % NBUF],
                                         agg_sh.at[sslab.at[j]],
                                         ssems[j % NBUF], add=True)
                if j >= 1:
                    sd[j - 1].wait()
                if j + 3 < SB:
                    b = (j + 3) % NBUF
                    gd[j + 3] = pltpu.async_copy(
                        ytab_ref.at[gslab.at[j + 3]], bufs[b], gsems[b])
            sd[SB - 1].wait()
            return carry

        lax.fori_loop(0, NBLK, blk_step, 0)
        plsc.subcore_barrier()
        pltpu.sync_copy(agg_sh.at[pl.ds(base, RPT)],
                        agg_out.at[c, pl.ds(base, RPT)])

    return pl.kernel(
        body,
        out_type=jax.ShapeDtypeStruct((NC, NP, D), jnp.float32),
        mesh=_mesh(),
        scratch_types=[
            pltpu.VMEM((SB, CH), jnp.int32),
            pltpu.VMEM((SB, CH), jnp.int32),
        ] + [pltpu.VMEM((CH, D), jnp.float32) for _ in range(NBUF)]
        + [
            pltpu.VMEM((ZR, D), jnp.float32),
            pltpu.VMEM_SHARED((NP, D), jnp.float32),
        ] + [pltpu.SemaphoreType.DMA for _ in range(2 * NBUF)],
    )(ytab, gidx, sidx)


SBC = 16


def _counts(sidxc):
    def body(sidx_ref, cnt_out, sslab, ones, zrow, cnt_sh, sem):
        c = lax.axis_index("c")
        s = lax.axis_index("s")

        z16 = jnp.zeros((16,), jnp.float32)
        o16 = jnp.ones((16,), jnp.float32)
        for i in range(ZR):
            for k in range(D // 16):
                zrow[i, pl.ds(k * 16, 16)] = z16
        for i in range(2 * CH):
            for k in range(D // 16):
                ones[i, pl.ds(k * 16, 16)] = o16

        base = s * RPT
        for t in range(RPT // ZR):
            pltpu.sync_copy(zrow, cnt_sh.at[pl.ds(base + t * ZR, ZR)])
        plsc.subcore_barrier()

        def blk_step(blk, carry):
            pltpu.sync_copy(sidx_ref.at[c, s, blk], sslab)
            sd = [pltpu.async_copy(ones, cnt_sh.at[sslab.at[j]], sem,
                                   add=True)
                  for j in range(SBC)]
            for d in sd:
                d.wait()
            return carry

        lax.fori_loop(0, NBLK, blk_step, 0)
        plsc.subcore_barrier()
        pltpu.sync_copy(cnt_sh.at[pl.ds(base, RPT)],
                        cnt_out.at[c, pl.ds(base, RPT)])

    return pl.kernel(
        body,
        out_type=jax.ShapeDtypeStruct((NC, NP, D), jnp.float32),
        mesh=_mesh(),
        scratch_types=[
            pltpu.VMEM((SBC, 2 * CH), jnp.int32),
            pltpu.VMEM((2 * CH, D), jnp.float32),
            pltpu.VMEM((ZR, D), jnp.float32),
            pltpu.VMEM_SHARED((NP, D), jnp.float32),
            pltpu.SemaphoreType.DMA,
        ],
    )(sidxc)


_NB = 10
_BLK = N // _NB


def _tc_pre(x, wfl, wbl, wfr, wbr):
    def body(x_ref, wfl_ref, wbl_ref, wfr_ref, wbr_ref, y_ref, r_ref):
        xb = x_ref[...]
        y_ref[0] = _dot(xb, wfl_ref[...])
        y_ref[1] = _dot(xb, wbl_ref[...])
        r_ref[0] = _dot(xb, wfr_ref[...])
        r_ref[1] = _dot(xb, wbr_ref[...])

    w_spec = pl.BlockSpec((D, D), lambda i: (0, 0))
    return pl.pallas_call(
        body,
        grid=(_NB,),
        in_specs=[pl.BlockSpec((_BLK, D), lambda i: (i, 0)),
                  w_spec, w_spec, w_spec, w_spec],
        out_specs=[pl.BlockSpec((2, _BLK, D), lambda i: (0, i, 0)),
                   pl.BlockSpec((2, _BLK, D), lambda i: (0, i, 0))],
        out_shape=[jax.ShapeDtypeStruct((2, N, D), jnp.float32),
                   jax.ShapeDtypeStruct((2, N, D), jnp.float32)],
    )(x, wfl, wbl, wfr, wbr)


def _tc_stats(agg, cnt, rtab, bf, bb):
    def body(agg_ref, cnt_ref, rtab_ref, bf_ref, bb_ref, h_ref, s_ref, q_ref):
        i = pl.program_id(0)
        cin = jnp.maximum(cnt_ref[0, :, 0:1], 1.0)
        cout = jnp.maximum(cnt_ref[1, :, 0:1], 1.0)
        hf = agg_ref[0] / cin + bf_ref[...] + rtab_ref[0]
        hb = agg_ref[1] / cout + bb_ref[...] + rtab_ref[1]
        h = jnp.concatenate([hf, hb], axis=1)
        h_ref[...] = h

        @pl.when(i == 0)
        def _():
            s_ref[...] = jnp.zeros_like(s_ref)
            q_ref[...] = jnp.zeros_like(q_ref)

        s_ref[...] += jnp.sum(h, axis=0, keepdims=True)
        q_ref[...] += jnp.sum(h * h, axis=0, keepdims=True)

    b_spec = pl.BlockSpec((1, D), lambda i: (0, 0))
    stat_spec = pl.BlockSpec((1, HID), lambda i: (0, 0))
    return pl.pallas_call(
        body,
        grid=(_NB,),
        in_specs=[pl.BlockSpec((2, _BLK, D), lambda i: (0, i, 0)),
                  pl.BlockSpec((2, _BLK, D), lambda i: (0, i, 0)),
                  pl.BlockSpec((2, _BLK, D), lambda i: (0, i, 0)),
                  b_spec, b_spec],
        out_specs=[pl.BlockSpec((_BLK, HID), lambda i: (i, 0)),
                   stat_spec, stat_spec],
        out_shape=[jax.ShapeDtypeStruct((N, HID), jnp.float32),
                   jax.ShapeDtypeStruct((1, HID), jnp.float32),
                   jax.ShapeDtypeStruct((1, HID), jnp.float32)],
    )(agg, cnt, rtab, bf, bb)


def _bn_relu(h, s, q, g, b):
    m = s / N
    v = q / N - m * m
    return jnp.maximum((h - m) * lax.rsqrt(v + EPS) * g + b, 0.0)


def _tc_apply_mid(h, s, q, g, b, wfl, wbl, wfr, wbr):
    def body(h_ref, s_ref, q_ref, g_ref, b_ref,
             wfl_ref, wbl_ref, wfr_ref, wbr_ref, y2_ref, r2_ref):
        hn = _bn_relu(h_ref[...], s_ref[...], q_ref[...],
                      g_ref[...], b_ref[...])
        y2_ref[0] = _dot(hn, wfl_ref[...])
        y2_ref[1] = _dot(hn, wbl_ref[...])
        r2_ref[0] = _dot(hn, wfr_ref[...])
        r2_ref[1] = _dot(hn, wbr_ref[...])

    stat_spec = pl.BlockSpec((1, HID), lambda i: (0, 0))
    w_spec = pl.BlockSpec((HID, D), lambda i: (0, 0))
    return pl.pallas_call(
        body,
        grid=(_NB,),
        in_specs=[pl.BlockSpec((_BLK, HID), lambda i: (i, 0)),
                  stat_spec, stat_spec, stat_spec, stat_spec,
                  w_spec, w_spec, w_spec, w_spec],
        out_specs=[pl.BlockSpec((2, _BLK, D), lambda i: (0, i, 0)),
                   pl.BlockSpec((2, _BLK, D), lambda i: (0, i, 0))],
        out_shape=[jax.ShapeDtypeStruct((2, N, D), jnp.float32),
                   jax.ShapeDtypeStruct((2, N, D), jnp.float32)],
    )(h, s, q, g, b, wfl, wbl, wfr, wbr)


def _tc_apply_post(h, s, q, g, b, wp, bp):
    def body(h_ref, s_ref, q_ref, g_ref, b_ref, wp_ref, bp_ref,
             embs_ref, genc_ref):
        i = pl.program_id(0)
        embs = _bn_relu(h_ref[...], s_ref[...], q_ref[...],
                        g_ref[...], b_ref[...])
        embs_ref[...] = embs
        proj = jnp.maximum(_dot(embs, wp_ref[...]) + bp_ref[...], 0.0)
        pmax = jnp.max(proj, axis=0, keepdims=True)

        @pl.when(i == 0)
        def _():
            genc_ref[...] = jnp.zeros_like(genc_ref)

        genc_ref[...] = jnp.maximum(genc_ref[...], pmax)

    stat_spec = pl.BlockSpec((1, HID), lambda i: (0, 0))
    return pl.pallas_call(
        body,
        grid=(_NB,),
        in_specs=[pl.BlockSpec((_BLK, HID), lambda i: (i, 0)),
                  stat_spec, stat_spec, stat_spec, stat_spec,
                  pl.BlockSpec((HID, HID), lambda i: (0, 0)),
                  stat_spec],
        out_specs=[pl.BlockSpec((_BLK, HID), lambda i: (i, 0)),
                   stat_spec],
        out_shape=[jax.ShapeDtypeStruct((N, HID), jnp.float32),
                   jax.ShapeDtypeStruct((1, HID), jnp.float32)],
    )(h, s, q, g, b, wp, bp)


def kernel(x, edge_index, W1f_l, b1f, W1f_r, W1b_l, b1b, W1b_r, bn1_g, bn1_b,
           W2f_l, b2f, W2f_r, W2b_l, b2b, W2b_r, bn2_g, bn2_b, Wp, bp):
    src = edge_index[0]
    dst = edge_index[1]
    pad = EPC - E
    zpad = jnp.zeros((pad,), jnp.int32)
    tpad = jnp.full((pad,), N, jnp.int32)
    gidx = jnp.stack([jnp.concatenate([src, zpad]),
                      jnp.concatenate([dst + N, zpad])])
    sidx = jnp.stack([jnp.concatenate([dst, tpad]),
                      jnp.concatenate([src, tpad])])
    gidx = gidx.reshape(NC, NS, NBLK, SB, CH)
    sidx = sidx.reshape(NC, NS, NBLK, SB, CH)

    cnt = _counts(sidx.reshape(NC, NS, NBLK, SBC, 2 * CH))
    ytab1, rtab1 = _tc_pre(x, W1f_l, W1b_l, W1f_r, W1b_r)
    agg1 = _aggregate(ytab1.reshape(NC * N, D), gidx, sidx)
    h1, s1, q1 = _tc_stats(agg1, cnt, rtab1,
                           b1f.reshape(1, D), b1b.reshape(1, D))
    ytab2, rtab2 = _tc_apply_mid(h1, s1, q1,
                                 bn1_g.reshape(1, HID), bn1_b.reshape(1, HID),
                                 W2f_l, W2b_l, W2f_r, W2b_r)
    agg2 = _aggregate(ytab2.reshape(NC * N, D), gidx, sidx)
    h2, s2, q2 = _tc_stats(agg2, cnt, rtab2,
                           b2f.reshape(1, D), b2b.reshape(1, D))
    node_embs, genc = _tc_apply_post(h2, s2, q2,
                                     bn2_g.reshape(1, HID),
                                     bn2_b.reshape(1, HID),
                                     Wp, bp.reshape(1, HID))
    return (node_embs, genc.reshape(HID))
